# baseline (speedup 1.0000x reference)
import functools

import jax
import jax.numpy as jnp
import numpy as np
from jax import lax
from jax.experimental import pallas as pl
from jax.experimental.pallas import tpu as pltpu
from jax.experimental.pallas import tpu_sc as plsc

L_LV = 14
F = 2
T = 2 ** 19
N_MIN = 32
N_MAX = 2048
GROWTH = (N_MAX / N_MIN) ** (1.0 / (L_LV - 1))

P1_I32 = -1640531535
P2_I32 = 805459861

NC = 2
NS = 16
NW = NC * NS
LANES = 16

_LEVELS = []
for _l in range(L_LV):
    _scale = N_MIN * (GROWTH ** _l) - 1.0
    _res = int(np.ceil(_scale)) + 1
    _LEVELS.append((float(_scale), _res, (_res ** 3) > T))

BP = 1024
NG = BP // LANES

_CORNERS = [(cx, cy, cz) for cx in (0, 1) for cy in (0, 1) for cz in (0, 1)]


def _sc_encode_kernel(n_pts):
    ppw = n_pts // NW
    nblk = ppw // BP
    mesh = plsc.VectorSubcoreMesh(core_axis_name="c", subcore_axis_name="s",
                                  num_cores=NC, num_subcores=NS)

    @functools.partial(
        pl.kernel,
        out_type=jax.ShapeDtypeStruct((n_pts * L_LV * F,), jnp.float32),
        mesh=mesh,
        compiler_params=pltpu.CompilerParams(needs_layout_passes=False,
                                             use_tc_tiling_on_sc=False),
        scratch_types=[
            pltpu.VMEM((3, BP), jnp.float32),
            pltpu.VMEM((8 * BP,), jnp.int32),
            pltpu.VMEM((8 * BP,), jnp.float32),
            pltpu.VMEM((8 * BP, F), jnp.float32),
            pltpu.VMEM((BP * L_LV * F,), jnp.float32),
        ],
    )
    def enc_kernel(xyz_ref, tab_ref, enc_ref, xyz_v, idx_v, w_v, rows_v, enc_v):
        wid = lax.axis_index("s") * NC + lax.axis_index("c")
        iota = lax.iota(jnp.int32, LANES)
        col0 = jnp.zeros((LANES,), jnp.int32)
        col1 = jnp.ones((LANES,), jnp.int32)

        def block_body(blk, carry):
            pbase = wid * ppw + blk * BP
            pltpu.sync_copy(xyz_ref.at[:, pl.ds(pbase, BP)], xyz_v)

            for l, (scale, res, use_hash) in enumerate(_LEVELS):
                a = np.float32(scale * 0.5)
                b = np.float32(scale * 0.5 + 0.5)
                lt = l * T

                def ph1(g, c2, a=a, b=b, lt=lt, res=res, use_hash=use_hash):
                    p = g * LANES
                    x = xyz_v[0, pl.ds(p, LANES)]
                    y = xyz_v[1, pl.ds(p, LANES)]
                    z = xyz_v[2, pl.ds(p, LANES)]
                    px = x * a + b
                    py = y * a + b
                    pz = z * a + b
                    bx = px.astype(jnp.int32)
                    by = py.astype(jnp.int32)
                    bz = pz.astype(jnp.int32)
                    fx = px - bx.astype(jnp.float32)
                    fy = py - by.astype(jnp.float32)
                    fz = pz - bz.astype(jnp.float32)
                    if use_hash:
                        hx = (bx, bx + 1)
                        hy0 = by * P1_I32
                        hy = (hy0, hy0 + P1_I32)
                        hz0 = bz * P2_I32
                        hz = (hz0, hz0 + P2_I32)

                        def cindex(cx, cy, cz):
                            return (((hx[cx] ^ hy[cy]) ^ hz[cz]) & (T - 1)) + lt
                    else:
                        r1 = res - 1
                        ix = (jnp.minimum(bx, r1), jnp.minimum(bx + 1, r1))
                        iy0 = jnp.minimum(by, r1) * res
                        iy = (iy0, jnp.minimum(by + 1, r1) * res)
                        iz0 = jnp.minimum(bz, r1) * (res * res) + lt
                        iz = (iz0, jnp.minimum(bz + 1, r1) * (res * res) + lt)

                        def cindex(cx, cy, cz):
                            return (ix[cx] + iy[cy]) + iz[cz]

                    wx = (1.0 - fx, fx)
                    wy = (1.0 - fy, fy)
                    wz = (1.0 - fz, fz)
                    wxy = {}
                    for cx in (0, 1):
                        for cy in (0, 1):
                            wxy[(cx, cy)] = wx[cx] * wy[cy]
                    for c, (cx, cy, cz) in enumerate(_CORNERS):
                        off = c * BP + p
                        idx_v[pl.ds(off, LANES)] = cindex(cx, cy, cz)
                        w_v[pl.ds(off, LANES)] = wxy[(cx, cy)] * wz[cz]
                    return c2

                lax.fori_loop(0, NG, ph1, 0, unroll=False)

                pltpu.sync_copy(tab_ref.at[idx_v], rows_v)

                def ph3(g, c2, l=l):
                    p = g * LANES
                    flat = iota + p
                    f0 = jnp.zeros((LANES,), jnp.float32)
                    f1 = jnp.zeros((LANES,), jnp.float32)
                    for c in range(8):
                        off = c * BP + p
                        r = flat + (c * BP)
                        g0 = plsc.load_gather(rows_v, [r, col0])
                        g1 = plsc.load_gather(rows_v, [r, col1])
                        w = w_v[pl.ds(off, LANES)]
                        f0 = f0 + w * g0
                        f1 = f1 + w * g1
                    epos = flat * (L_LV * F) + (2 * l)
                    plsc.store_scatter(enc_v, [epos], f0)
                    plsc.store_scatter(enc_v, [epos + 1], f1)
                    return c2

                lax.fori_loop(0, NG, ph3, 0, unroll=False)

            pltpu.sync_copy(enc_v, enc_ref.at[pl.ds(pbase * (L_LV * F), BP * L_LV * F)])
            return carry

        lax.fori_loop(0, nblk, block_body, 0, unroll=False)

    return enc_kernel


def _mlp_body(enc_ref, w1_ref, m_ref, b1_ref, w2_ref, b2_ref, out_ref):
    x = enc_ref[...]
    w1 = w1_ref[...] * m_ref[...]
    h = jnp.dot(x, w1, preferred_element_type=jnp.float32) + b1_ref[...]
    h = jnp.maximum(h, 0.0)
    out_ref[...] = jnp.dot(h, w2_ref[...], preferred_element_type=jnp.float32) + b2_ref[...]


def kernel(xyz, tables, masking, W1, b1, W2, b2):
    n_pts = xyz.shape[0]
    xyz_t = xyz.T
    tab = tables.reshape(L_LV * T, F)
    enc = _sc_encode_kernel(n_pts)(xyz_t, tab).reshape(n_pts, L_LV * F)

    bt = 4096
    kd = L_LV * F
    width = W1.shape[1]
    ch = W2.shape[1]
    out = pl.pallas_call(
        _mlp_body,
        grid=(n_pts // bt,),
        in_specs=[
            pl.BlockSpec((bt, kd), lambda i: (i, 0)),
            pl.BlockSpec((kd, width), lambda i: (0, 0)),
            pl.BlockSpec((kd, 1), lambda i: (0, 0)),
            pl.BlockSpec((1, width), lambda i: (0, 0)),
            pl.BlockSpec((width, ch), lambda i: (0, 0)),
            pl.BlockSpec((1, ch), lambda i: (0, 0)),
        ],
        out_specs=pl.BlockSpec((bt, ch), lambda i: (i, 0)),
        out_shape=jax.ShapeDtypeStruct((n_pts, ch), jnp.float32),
    )(enc, W1, masking.reshape(kd, 1), b1.reshape(1, width), W2, b2.reshape(1, ch))
    return out


# speedup vs baseline: 7.2975x; 7.2975x over previous
import functools

import jax
import jax.numpy as jnp
import numpy as np
from jax import lax
from jax.experimental import pallas as pl
from jax.experimental.pallas import tpu as pltpu
from jax.experimental.pallas import tpu_sc as plsc

L_LV = 14
F = 2
T = 2 ** 19
N_MIN = 32
N_MAX = 2048
GROWTH = (N_MAX / N_MIN) ** (1.0 / (L_LV - 1))

P1_I32 = -1640531535
P2_I32 = 805459861

NC = 2
NS = 16
NW = NC * NS
LANES = 16

_LEVELS = []
for _l in range(L_LV):
    _scale = N_MIN * (GROWTH ** _l) - 1.0
    _res = int(np.ceil(_scale)) + 1
    _LEVELS.append((float(_scale), _res, (_res ** 3) > T))

BP = 512
NG = BP // LANES
RP = 8

_CORNERS = [(cx, cy, cz) for cx in (0, 1) for cy in (0, 1) for cz in (0, 1)]

_NBLOCKS = L_LV * T // 128
_BLK_PER_W = _NBLOCKS // NW
_CHUNK_BLKS = 16

_OCT_CHUNK = 2048
_OCT_INFO = []
_ob = 0
for _l, (_s, _res, _h) in enumerate(_LEVELS):
    if not _h:
        _nch = -(-(_res ** 3) // _OCT_CHUNK)
        _OCT_INFO.append((_l, _res, _l * T, _ob, _nch))
        _ob += _nch * _OCT_CHUNK
_OCT_N = _ob
_OCT_TOT_CHUNKS = _OCT_N // _OCT_CHUNK
_OCT_HALO = max(r * r + r + 1 for (_, r, _, _, _) in _OCT_INFO)
_OCT_IN_ROWS = _OCT_CHUNK + ((_OCT_HALO + 15) // 16) * 16


def _octtab_kernel():
    mesh = plsc.VectorSubcoreMesh(core_axis_name="c", subcore_axis_name="s",
                                  num_cores=NC, num_subcores=NS)
    nit = -(-_OCT_TOT_CHUNKS // NW)
    b1 = _OCT_INFO[0][4]
    b2 = b1 + _OCT_INFO[1][4]

    @functools.partial(
        pl.kernel,
        out_type=jax.ShapeDtypeStruct((_OCT_N * 16,), jnp.float32),
        mesh=mesh,
        compiler_params=pltpu.CompilerParams(needs_layout_passes=False,
                                             use_tc_tiling_on_sc=False),
        scratch_types=[
            pltpu.VMEM((_OCT_IN_ROWS * F,), jnp.float32),
            pltpu.VMEM((_OCT_CHUNK * 16,), jnp.float32),
        ],
    )
    def octk(tabf_ref, out_ref, in_v, out_v):
        wid = lax.axis_index("s") * NC + lax.axis_index("c")
        iota = lax.iota(jnp.int32, LANES)

        def it_body(it, carry):
            cid = it * NW + wid

            @pl.when(cid < _OCT_TOT_CHUNKS)
            def _():
                in1 = cid >= b1
                in2 = cid >= b2
                res = jnp.where(in2, _OCT_INFO[2][1],
                                jnp.where(in1, _OCT_INFO[1][1],
                                          _OCT_INFO[0][1]))
                std_base = jnp.where(in2, _OCT_INFO[2][2],
                                     jnp.where(in1, _OCT_INFO[1][2],
                                               _OCT_INFO[0][2]))
                oct_base = jnp.where(in2, _OCT_INFO[2][3],
                                     jnp.where(in1, _OCT_INFO[1][3],
                                               _OCT_INFO[0][3]))
                row0 = std_base + cid * _OCT_CHUNK - oct_base
                pltpu.sync_copy(
                    tabf_ref.at[pl.ds(row0 * F, _OCT_IN_ROWS * F)], in_v)
                res2 = res * res
                offs = [(q & 1) + ((q >> 1) & 1) * res + ((q >> 2) & 1) * res2
                        for q in range(8)]

                def grp(g, c2):
                    base = iota + g * LANES
                    base16 = base * 16
                    for q in range(8):
                        pos = (base + offs[q]) * F
                        g0 = plsc.load_gather(in_v, [pos])
                        g1 = plsc.load_gather(in_v, [pos + 1])
                        plsc.store_scatter(out_v, [base16 + (2 * q)], g0)
                        plsc.store_scatter(out_v, [base16 + (2 * q + 1)], g1)
                    return c2

                lax.fori_loop(0, _OCT_CHUNK // LANES, grp, 0, unroll=False)
                pltpu.sync_copy(
                    out_v,
                    out_ref.at[pl.ds(cid * (_OCT_CHUNK * 16),
                                     _OCT_CHUNK * 16)])

            return carry

        lax.fori_loop(0, nit, it_body, 0, unroll=False)

    return octk


def _relayout_kernel():
    mesh = plsc.VectorSubcoreMesh(core_axis_name="c", subcore_axis_name="s",
                                  num_cores=NC, num_subcores=NS)

    @functools.partial(
        pl.kernel,
        out_type=jax.ShapeDtypeStruct((L_LV * T * F,), jnp.float32),
        mesh=mesh,
        compiler_params=pltpu.CompilerParams(needs_layout_passes=False,
                                             use_tc_tiling_on_sc=False),
        scratch_types=[
            pltpu.VMEM((_CHUNK_BLKS * 256,), jnp.float32),
            pltpu.VMEM((_CHUNK_BLKS * 256,), jnp.float32),
        ],
    )
    def relayout(tabn_ref, out_ref, in_v, out_v):
        wid = lax.axis_index("s") * NC + lax.axis_index("c")
        iota2 = lax.iota(jnp.int32, LANES) * 2
        nchunks = _BLK_PER_W // _CHUNK_BLKS
        blk0 = wid * _BLK_PER_W

        def chunk_body(kc, carry):
            b0 = blk0 + kc * _CHUNK_BLKS
            pltpu.sync_copy(tabn_ref.at[pl.ds(b0 * 256, _CHUNK_BLKS * 256)],
                            in_v)

            def blk_body(m, c2):
                for v in range(8):
                    f0 = in_v[pl.ds(m * 256 + v * LANES, LANES)]
                    f1 = in_v[pl.ds(m * 256 + 128 + v * LANES, LANES)]
                    opos = iota2 + (m * 256 + v * (LANES * 2))
                    plsc.store_scatter(out_v, [opos], f0)
                    plsc.store_scatter(out_v, [opos + 1], f1)
                return c2

            lax.fori_loop(0, _CHUNK_BLKS, blk_body, 0, unroll=False)
            pltpu.sync_copy(
                out_v,
                out_ref.at[pl.ds(b0 * 256, _CHUNK_BLKS * 256)])
            return carry

        lax.fori_loop(0, nchunks, chunk_body, 0, unroll=False)

    return relayout


def _sc_encode_kernel(n_pts):
    ppw = n_pts // NW
    nblk = ppw // BP
    mesh = plsc.VectorSubcoreMesh(core_axis_name="c", subcore_axis_name="s",
                                  num_cores=NC, num_subcores=NS)

    @functools.partial(
        pl.kernel,
        out_type=jax.ShapeDtypeStruct((4, n_pts // 128, 1024), jnp.float32),
        mesh=mesh,
        compiler_params=pltpu.CompilerParams(needs_layout_passes=False,
                                             use_tc_tiling_on_sc=False),
        scratch_types=[
            pltpu.VMEM((3, BP), jnp.float32),
            pltpu.VMEM((8 * BP,), jnp.int32),
            pltpu.VMEM((8 * BP,), jnp.int32),
            pltpu.VMEM((8 * BP,), jnp.float32),
            pltpu.VMEM((8 * BP,), jnp.float32),
            pltpu.VMEM((8 * BP, RP), jnp.float32),
            pltpu.VMEM((8 * BP, RP), jnp.float32),
            pltpu.VMEM((BP,), jnp.int32),
            pltpu.VMEM((BP,), jnp.int32),
            pltpu.VMEM((BP,), jnp.int32),
            pltpu.VMEM((BP,), jnp.int32),
            pltpu.VMEM((BP, 16), jnp.float32),
            pltpu.VMEM((BP, 16), jnp.float32),
            pltpu.VMEM((BP,), jnp.int32),
            pltpu.VMEM((BP,), jnp.int32),
            pltpu.VMEM((4, BP // 128, 1024), jnp.float32),
            pltpu.SemaphoreType.DMA,
            pltpu.SemaphoreType.DMA,
            pltpu.SemaphoreType.DMA,
            pltpu.SemaphoreType.DMA,
        ],
    )
    def enc_kernel(xyz_ref, tab_ref, oct_ref, enc_ref, xyz_v, idx0_v, idx1_v,
                   w0_v, w1_v, rows0_v, rows1_v, oi0_v, oi1_v, d0_v, d1_v,
                   orows0_v, orows1_v, lb0_v, lb1_v, enc_v,
                   sem0, sem1, sem2, sem3):
        wid = lax.axis_index("s") * NC + lax.axis_index("c")
        iota = lax.iota(jnp.int32, LANES)
        col0 = jnp.zeros((LANES,), jnp.int32)
        col1 = jnp.ones((LANES,), jnp.int32)
        idx_bufs = (idx0_v, idx1_v)
        w_bufs = (w0_v, w1_v)
        rows_bufs = (rows0_v, rows1_v)
        oi_bufs = (oi0_v, oi1_v)
        d_bufs = (d0_v, d1_v)
        orows_bufs = (orows0_v, orows1_v)
        lb_bufs = (lb0_v, lb1_v)
        sems = (sem0, sem1)
        semsD = (sem2, sem3)
        octbase = {info[0]: info[3] for info in _OCT_INFO}

        def store_enc(g, l2, f0, f1):
            rt = (2 * l2) // 8
            fr = (2 * l2) % 8
            lane0 = (g & 7) * LANES
            enc_v[rt, g >> 3, pl.ds(fr * 128 + lane0, LANES)] = f0
            enc_v[rt, g >> 3, pl.ds((fr + 1) * 128 + lane0, LANES)] = f1

        def ph3_hash(g, c2, l2, rows_v, w_v, lb_v):
            p = g * LANES
            flat = iota + p
            lb = lb_v[pl.ds(p, LANES)]
            sel4 = ((lb >> 16) & 1) * (4 * BP)
            f0 = jnp.zeros((LANES,), jnp.float32)
            f1 = jnp.zeros((LANES,), jnp.float32)
            for c in range(8):
                col = ((lb >> (2 * c)) & 3) * 2
                if c < 4:
                    r = flat + (c * BP)
                else:
                    r = flat + ((c - 4) * BP) + sel4
                g0 = plsc.load_gather(rows_v, [r, col])
                g1 = plsc.load_gather(rows_v, [r, col + 1])
                w = w_v[pl.ds(c * BP + p, LANES)]
                f0 = f0 + w * g0
                f1 = f1 + w * g1
            store_enc(g, l2, f0, f1)
            return c2

        def ph3_direct(g, c2, l2, orows_v, d_v, w_v):
            p = g * LANES
            flat = iota + p
            d = d_v[pl.ds(p, LANES)]
            f0 = jnp.zeros((LANES,), jnp.float32)
            f1 = jnp.zeros((LANES,), jnp.float32)
            for c, (cx, cy, cz) in enumerate(_CORNERS):
                m = cx + 2 * cy + 4 * cz
                if m == 0:
                    c0, c1 = col0, col1
                else:
                    c0 = (d & m) * 2
                    c1 = c0 + 1
                g0 = plsc.load_gather(orows_v, [flat, c0])
                g1 = plsc.load_gather(orows_v, [flat, c1])
                w = w_v[pl.ds(c * BP + p, LANES)]
                f0 = f0 + w * g0
                f1 = f1 + w * g1
            store_enc(g, l2, f0, f1)
            return c2

        def drain(pending):
            kind, pl_, pp_, ph_ = pending
            ph_.wait()
            if kind == 0:
                body = functools.partial(ph3_hash, l2=pl_,
                                         rows_v=rows_bufs[pp_],
                                         w_v=w_bufs[pp_], lb_v=lb_bufs[pp_])
            else:
                body = functools.partial(ph3_direct, l2=pl_,
                                         orows_v=orows_bufs[pp_],
                                         d_v=d_bufs[pp_], w_v=w_bufs[pp_])
            lax.fori_loop(0, NG, body, 0, unroll=False)

        def block_body(blk, carry):
            pbase = wid * ppw + blk * BP
            pltpu.sync_copy(xyz_ref.at[:, pl.ds(pbase, BP)], xyz_v)
            pending = None

            order = [i for i in range(L_LV) if _LEVELS[i][2]] + \
                [i for i in range(L_LV) if not _LEVELS[i][2]]
            for l in order:
                scale, res, use_hash = _LEVELS[l]
                a = np.float32(scale * 0.5)
                b = np.float32(scale * 0.5 + 0.5)
                lt = l * T
                par = l & 1
                idx_v = idx_bufs[par]
                w_v = w_bufs[par]
                oi_v = oi_bufs[par]
                d_v = d_bufs[par]
                lb_v = lb_bufs[par]
                ob = octbase.get(l, 0)
                ltq = l * T // 4

                def ph1(g, c2, a=a, b=b, ltq=ltq, ob=ob, res=res,
                        use_hash=use_hash, idx_v=idx_v, w_v=w_v, oi_v=oi_v,
                        d_v=d_v, lb_v=lb_v):
                    p = g * LANES
                    x = xyz_v[0, pl.ds(p, LANES)]
                    y = xyz_v[1, pl.ds(p, LANES)]
                    z = xyz_v[2, pl.ds(p, LANES)]
                    px = x * a + b
                    py = y * a + b
                    pz = z * a + b
                    bx = px.astype(jnp.int32)
                    by = py.astype(jnp.int32)
                    bz = pz.astype(jnp.int32)
                    fx = px - bx.astype(jnp.float32)
                    fy = py - by.astype(jnp.float32)
                    fz = pz - bz.astype(jnp.float32)
                    wx = (1.0 - fx, fx)
                    wy = (1.0 - fy, fy)
                    wz = (1.0 - fz, fz)
                    wxy = {}
                    for cx in (0, 1):
                        for cy in (0, 1):
                            wxy[(cx, cy)] = wx[cx] * wy[cy]
                    for c, (cx, cy, cz) in enumerate(_CORNERS):
                        w_v[pl.ds(c * BP + p, LANES)] = wxy[(cx, cy)] * wz[cz]
                    if use_hash:
                        hx = (bx, bx + 1)
                        hy0 = by * P1_I32
                        hy = (hy0, hy0 + P1_I32)
                        hz0 = bz * P2_I32
                        hz = (hz0, hz0 + P2_I32)
                        ncov = ((bx & 3) == 3).astype(jnp.int32)
                        lb = ncov << 16
                        for c, (cx, cy, cz) in enumerate(_CORNERS):
                            e = ((hx[cx] ^ hy[cy]) ^ hz[cz]) & (T - 1)
                            lb = lb | ((e & 3) << (2 * c))
                            q = (e >> 2) + ltq
                            if cx == 1:
                                q = jnp.where(ncov == 1, q, -1)
                            idx_v[pl.ds(c * BP + p, LANES)] = q
                        lb_v[pl.ds(p, LANES)] = lb
                    else:
                        r1 = res - 1
                        x0 = jnp.minimum(bx, r1)
                        y0 = jnp.minimum(by, r1)
                        z0 = jnp.minimum(bz, r1)
                        dx1 = jnp.minimum(bx + 1, r1) - x0
                        dy1 = jnp.minimum(by + 1, r1) - y0
                        dz1 = jnp.minimum(bz + 1, r1) - z0
                        oi_v[pl.ds(p, LANES)] = (x0 + y0 * res
                                                 + z0 * (res * res)) + ob
                        d_v[pl.ds(p, LANES)] = dx1 + dy1 * 2 + dz1 * 4
                    return c2

                lax.fori_loop(0, NG, ph1, 0, unroll=False)

                if use_hash:
                    dma = pltpu.async_copy(
                        tab_ref.at[plsc.Indices(idx_v, ignored_value=-1)],
                        rows_bufs[par], sems[par])
                    this = (0, l, par, dma)
                else:
                    dma = pltpu.async_copy(oct_ref.at[oi_v], orows_bufs[par],
                                           semsD[par])
                    this = (1, l, par, dma)

                if pending is not None:
                    drain(pending)
                pending = this

            drain(pending)
            zv = jnp.zeros((LANES,), jnp.float32)
            for ct in range(BP // 128):
                for fr in range(4, 8):
                    for jb in range(128 // LANES):
                        enc_v[3, ct, pl.ds(fr * 128 + jb * LANES, LANES)] = zv
            ctg0 = pbase >> 7
            pltpu.sync_copy(enc_v, enc_ref.at[:, pl.ds(ctg0, BP // 128), :])
            return carry

        lax.fori_loop(0, nblk, block_body, 0, unroll=False)

    return enc_kernel


def _mlp_body(enc_ref, w1_ref, m_ref, b1_ref, w2_ref, b2_ref, out_ref):
    x = enc_ref[...]
    w1 = w1_ref[...] * m_ref[...]
    h = lax.dot_general(x, w1, (((0,), (0,)), ((), ())),
                        preferred_element_type=jnp.float32) + b1_ref[...]
    h = jnp.maximum(h, 0.0)
    out_ref[...] = jnp.dot(h, w2_ref[...], preferred_element_type=jnp.float32) + b2_ref[...]


def kernel(xyz, tables, masking, W1, b1, W2, b2):
    n_pts = xyz.shape[0]
    xyz_t = xyz.T
    tabn = tables.reshape(L_LV, T // 128, 128, F).transpose(0, 1, 3, 2)
    tabn = tabn.reshape(L_LV * T * F)
    tabf = _relayout_kernel()(tabn)
    octf = _octtab_kernel()(tabf)
    tab = tabf.reshape(L_LV * T // 4, 8)
    octtab = octf.reshape(_OCT_N, 16)
    enc4 = _sc_encode_kernel(n_pts)(xyz_t, tab, octtab)
    enc32 = enc4.reshape(4, n_pts // 128, 8, 128).transpose(0, 2, 1, 3)
    enc32 = enc32.reshape(32, n_pts)

    bt = 4096
    kd = L_LV * F
    width = W1.shape[1]
    ch = W2.shape[1]
    w1p = jnp.pad(W1, ((0, 32 - kd), (0, 0)))
    mp = jnp.pad(masking.reshape(kd, 1), ((0, 32 - kd), (0, 0)))
    out = pl.pallas_call(
        _mlp_body,
        grid=(n_pts // bt,),
        in_specs=[
            pl.BlockSpec((32, bt), lambda i: (0, i)),
            pl.BlockSpec((32, width), lambda i: (0, 0)),
            pl.BlockSpec((32, 1), lambda i: (0, 0)),
            pl.BlockSpec((1, width), lambda i: (0, 0)),
            pl.BlockSpec((width, ch), lambda i: (0, 0)),
            pl.BlockSpec((1, ch), lambda i: (0, 0)),
        ],
        out_specs=pl.BlockSpec((bt, ch), lambda i: (i, 0)),
        out_shape=jax.ShapeDtypeStruct((n_pts, ch), jnp.float32),
    )(enc32, w1p, mp, b1.reshape(1, width), W2, b2.reshape(1, ch))
    return out
